# baseline (speedup 1.0000x reference)
import functools

import jax
import jax.numpy as jnp
from jax import lax
from jax.experimental import pallas as pl
from jax.experimental.pallas import tpu as pltpu
from jax.experimental.pallas import tpu_sc as plsc

NUM_USERS = 1000000
NUM_ITEMS = 1000000
EMBED_DIM = 32
BATCH = 16384

NC, NS, L = 2, 16, 16
NW = NC * NS
B_PER_W = BATCH // NW
CHUNK = 128
NCHUNK = B_PER_W // CHUNK


def _body(user_ref, item_ref, ut_ref, it_ref, out_ref,
          idx_u, idx_i, rows_u, rows_i, out_v, sem):
  wid = lax.axis_index("s") * NC + lax.axis_index("c")
  base = wid * B_PER_W

  pltpu.sync_copy(user_ref.at[wid], idx_u)
  pltpu.sync_copy(item_ref.at[wid], idx_i)

  copies = []
  for j in range(NCHUNK):
    copies.append(pltpu.async_copy(
        ut_ref.at[idx_u.at[j]], rows_u.at[pl.ds(j * CHUNK, CHUNK)], sem))
    copies.append(pltpu.async_copy(
        it_ref.at[idx_i.at[j]], rows_i.at[pl.ds(j * CHUNK, CHUNK)], sem))
  for cp in copies:
    cp.wait()

  lane = lax.iota(jnp.int32, L)

  def block(b2, _):
    rb = b2 * L
    rid = rb + lane
    acc = jnp.zeros((L,), jnp.float32)
    for d in range(EMBED_DIM):
      col = jnp.full((L,), d, jnp.int32)
      u = plsc.load_gather(rows_u, [rid, col])
      v = plsc.load_gather(rows_i, [rid, col])
      acc = acc + u * v
    out_v[pl.ds(rb, L)] = acc
    return _

  lax.fori_loop(0, B_PER_W // L, block, None)

  pltpu.sync_copy(out_v, out_ref.at[pl.ds(base, B_PER_W)])


@jax.jit
def _scores(user_r, item_r, user_table, item_table):
  mesh = plsc.VectorSubcoreMesh(core_axis_name="c", subcore_axis_name="s",
                                num_cores=NC, num_subcores=NS)
  return pl.kernel(
      _body,
      out_type=jax.ShapeDtypeStruct((BATCH,), jnp.float32),
      mesh=mesh,
      compiler_params=pltpu.CompilerParams(needs_layout_passes=False,
                                           use_tc_tiling_on_sc=False),
      scratch_types=[
          pltpu.VMEM((NCHUNK, CHUNK), jnp.int32),
          pltpu.VMEM((NCHUNK, CHUNK), jnp.int32),
          pltpu.VMEM((B_PER_W, EMBED_DIM), jnp.float32),
          pltpu.VMEM((B_PER_W, EMBED_DIM), jnp.float32),
          pltpu.VMEM((B_PER_W,), jnp.float32),
          pltpu.SemaphoreType.DMA,
      ],
  )(user_r, item_r, user_table, item_table)


def kernel(user, item, user_table, item_table):
  user_r = user.astype(jnp.int32).reshape(NW, NCHUNK, CHUNK)
  item_r = item.astype(jnp.int32).reshape(NW, NCHUNK, CHUNK)
  return _scores(user_r, item_r, user_table, item_table)


# speedup vs baseline: 5.9465x; 5.9465x over previous
import jax
import jax.numpy as jnp
from jax import lax
from jax.experimental import pallas as pl
from jax.experimental.pallas import tpu as pltpu
from jax.experimental.pallas import tpu_sc as plsc

NUM_USERS = 1000000
NUM_ITEMS = 1000000
EMBED_DIM = 32
BATCH = 16384

NC, NS, L = 2, 16, 16
NW = NC * NS
B_PER_W = BATCH // NW
C = 64
PASSES = B_PER_W // C


def _body(user_ref, item_ref, ut_ref, it_ref, out_ref,
          idx_u, idx_i, buf_u, buf_i, out_v, sem):
  wid = lax.axis_index("s") * NC + lax.axis_index("c")
  base = wid * B_PER_W

  pltpu.sync_copy(user_ref.at[pl.ds(base, B_PER_W)], idx_u)
  pltpu.sync_copy(item_ref.at[pl.ds(base, B_PER_W)], idx_i)

  lane = lax.iota(jnp.int32, L)

  def one_pass(p, _):
    pb = p * C

    def fire_group(gg, _):
      vu = idx_u[pl.ds(pb + gg * L, L)]
      vi = idx_i[pl.ds(pb + gg * L, L)]
      for t in range(L):
        dst = (gg * L + t) * L
        ru = (vu[t] // L) * L
        ri = (vi[t] // L) * L
        pltpu.async_copy(ut_ref.at[:, :, pl.ds(ru, L)],
                         buf_u.at[:, :, pl.ds(dst, L)], sem)
        pltpu.async_copy(it_ref.at[:, :, pl.ds(ri, L)],
                         buf_i.at[:, :, pl.ds(dst, L)], sem)
      for t in range(L):
        pltpu.make_async_copy(ut_ref.at[:, :, pl.ds(0, L)],
                              buf_u.at[:, :, pl.ds(0, L)], sem).wait()
        pltpu.make_async_copy(it_ref.at[:, :, pl.ds(0, L)],
                              buf_i.at[:, :, pl.ds(0, L)], sem).wait()
      return _

    lax.fori_loop(0, C // L, fire_group, None)

    def block(blk, _):
      b0 = pb + blk * L
      vu = idx_u[pl.ds(b0, L)]
      vi = idx_i[pl.ds(b0, L)]
      lidx_u = (blk * L + lane) * L + (vu % L)
      lidx_i = (blk * L + lane) * L + (vi % L)
      acc = jnp.zeros((L,), jnp.float32)
      for i in range(4):
        ii = jnp.full((L,), i, jnp.int32)
        for s in range(8):
          ss = jnp.full((L,), s, jnp.int32)
          u = plsc.load_gather(buf_u, [ii, ss, lidx_u])
          v = plsc.load_gather(buf_i, [ii, ss, lidx_i])
          acc = acc + u * v
      out_v[pl.ds(b0, L)] = acc
      return _

    lax.fori_loop(0, C // L, block, None)
    return _

  lax.fori_loop(0, PASSES, one_pass, None)

  pltpu.sync_copy(out_v, out_ref.at[pl.ds(base, B_PER_W)])


@jax.jit
def _scores(user_r, item_r, ut3, it3):
  mesh = plsc.VectorSubcoreMesh(core_axis_name="c", subcore_axis_name="s",
                                num_cores=NC, num_subcores=NS)
  return pl.kernel(
      _body,
      out_type=jax.ShapeDtypeStruct((BATCH,), jnp.float32),
      mesh=mesh,
      compiler_params=pltpu.CompilerParams(needs_layout_passes=False,
                                           use_tc_tiling_on_sc=True),
      scratch_types=[
          pltpu.VMEM((B_PER_W,), jnp.int32),
          pltpu.VMEM((B_PER_W,), jnp.int32),
          pltpu.VMEM((4, 8, C * L), jnp.float32),
          pltpu.VMEM((4, 8, C * L), jnp.float32),
          pltpu.VMEM((B_PER_W,), jnp.float32),
          pltpu.SemaphoreType.DMA,
      ],
  )(user_r, item_r, ut3, it3)


def kernel(user, item, user_table, item_table):
  ut3 = user_table.T.reshape(4, 8, NUM_USERS)
  it3 = item_table.T.reshape(4, 8, NUM_ITEMS)
  return _scores(user.astype(jnp.int32), item.astype(jnp.int32), ut3, it3)


# speedup vs baseline: 6.6239x; 1.1139x over previous
import jax
import jax.numpy as jnp
from jax import lax
from jax.experimental import pallas as pl
from jax.experimental.pallas import tpu as pltpu
from jax.experimental.pallas import tpu_sc as plsc

NUM_USERS = 1000000
NUM_ITEMS = 1000000
EMBED_DIM = 32
BATCH = 16384

NC, NS, L = 2, 16, 16
NW = NC * NS
B_PER_W = BATCH // NW
C = 64
PASSES = B_PER_W // C


def _body(user_ref, item_ref, ut_ref, it_ref, out_ref,
          idx_u, idx_i, buf_u, buf_i, out_v, sem):
  wid = lax.axis_index("s") * NC + lax.axis_index("c")
  base = wid * B_PER_W

  pltpu.sync_copy(user_ref.at[pl.ds(base, B_PER_W)], idx_u)
  pltpu.sync_copy(item_ref.at[pl.ds(base, B_PER_W)], idx_i)

  lane = lax.iota(jnp.int32, L)

  def one_pass(p, _):
    pb = p * C

    def fire_group(gg, _):
      vu = idx_u[pl.ds(pb + gg * L, L)]
      vi = idx_i[pl.ds(pb + gg * L, L)]
      for t in range(L):
        dst = (gg * L + t) * L
        ru = (vu[t] // L) * L
        ri = (vi[t] // L) * L
        pltpu.async_copy(ut_ref.at[:, :, pl.ds(ru, L)],
                         buf_u.at[:, :, pl.ds(dst, L)], sem)
        pltpu.async_copy(it_ref.at[:, :, pl.ds(ri, L)],
                         buf_i.at[:, :, pl.ds(dst, L)], sem)
      return _

    lax.fori_loop(0, C // L, fire_group, None)

    def drain_group(gg, _):
      for t in range(L):
        pltpu.make_async_copy(ut_ref.at[:, :, pl.ds(0, L)],
                              buf_u.at[:, :, pl.ds(0, L)], sem).wait()
        pltpu.make_async_copy(it_ref.at[:, :, pl.ds(0, L)],
                              buf_i.at[:, :, pl.ds(0, L)], sem).wait()
      return _

    lax.fori_loop(0, C // L, drain_group, None)

    def block(blk, _):
      b0 = pb + blk * L
      vu = idx_u[pl.ds(b0, L)]
      vi = idx_i[pl.ds(b0, L)]
      lidx_u = (blk * L + lane) * L + (vu % L)
      lidx_i = (blk * L + lane) * L + (vi % L)
      acc = jnp.zeros((L,), jnp.float32)
      for i in range(4):
        ii = jnp.full((L,), i, jnp.int32)
        for s in range(8):
          ss = jnp.full((L,), s, jnp.int32)
          u = plsc.load_gather(buf_u, [ii, ss, lidx_u])
          v = plsc.load_gather(buf_i, [ii, ss, lidx_i])
          acc = acc + u * v
      out_v[pl.ds(b0, L)] = acc
      return _

    lax.fori_loop(0, C // L, block, None)
    return _

  lax.fori_loop(0, PASSES, one_pass, None)

  pltpu.sync_copy(out_v, out_ref.at[pl.ds(base, B_PER_W)])


@jax.jit
def _scores(user_r, item_r, ut3, it3):
  mesh = plsc.VectorSubcoreMesh(core_axis_name="c", subcore_axis_name="s",
                                num_cores=NC, num_subcores=NS)
  return pl.kernel(
      _body,
      out_type=jax.ShapeDtypeStruct((BATCH,), jnp.float32),
      mesh=mesh,
      compiler_params=pltpu.CompilerParams(needs_layout_passes=False,
                                           use_tc_tiling_on_sc=True),
      scratch_types=[
          pltpu.VMEM((B_PER_W,), jnp.int32),
          pltpu.VMEM((B_PER_W,), jnp.int32),
          pltpu.VMEM((4, 8, C * L), jnp.float32),
          pltpu.VMEM((4, 8, C * L), jnp.float32),
          pltpu.VMEM((B_PER_W,), jnp.float32),
          pltpu.SemaphoreType.DMA,
      ],
  )(user_r, item_r, ut3, it3)


def kernel(user, item, user_table, item_table):
  ut3 = user_table.T.reshape(4, 8, NUM_USERS)
  it3 = item_table.T.reshape(4, 8, NUM_ITEMS)
  return _scores(user.astype(jnp.int32), item.astype(jnp.int32), ut3, it3)
